# baseline (speedup 1.0000x reference)
import math

import jax
import jax.numpy as jnp
from jax import lax
from jax.experimental import pallas as pl
from jax.experimental.pallas import tpu as pltpu
from jax.experimental.pallas import tpu_sc as plsc

N = 10000
E = 320000
NP = 10240
NW = 32
WIN = NP // NW
ACC_R = WIN + 8
PKDUM = WIN
EPW = E // NW
CHUNK = 80
NCHUNK = EPW // CHUNK
GPT = E // 16
SUPG = 40
SUPW = SUPG * 16
NSUP = GPT // SUPG
FIRE = 32
PCAP = 176
PSHIFT = PCAP - FIRE
HID = 256
ALPHA = 0.5
BE = 512
BN = 512


def _mesh():
    return plsc.VectorSubcoreMesh(core_axis_name="c", subcore_axis_name="s")


def _gather_edges_call(nf_p, src, dst):
    def body(nf_hbm, src_hbm, dst_hbm, hs_hbm, hd_hbm, srcv, dstv, nbuf, sem):
        wid = lax.axis_index("s") * 2 + lax.axis_index("c")

        def step(i, carry):
            base = wid * EPW + i * CHUNK
            pltpu.sync_copy(src_hbm.at[pl.ds(base, CHUNK)], srcv)
            pltpu.sync_copy(dst_hbm.at[pl.ds(base, CHUNK)], dstv)
            pltpu.async_copy(nf_hbm.at[srcv], nbuf, sem).wait()
            pltpu.sync_copy(nbuf, hs_hbm.at[pl.ds(base, CHUNK)])
            pltpu.async_copy(nf_hbm.at[dstv], nbuf, sem).wait()
            pltpu.sync_copy(nbuf, hd_hbm.at[pl.ds(base, CHUNK)])
            return carry

        lax.fori_loop(0, NCHUNK, step, 0)

    f = pl.kernel(
        body,
        mesh=_mesh(),
        out_type=(
            jax.ShapeDtypeStruct((E, 128), jnp.float32),
            jax.ShapeDtypeStruct((E, 128), jnp.float32),
        ),
        scratch_types=[
            pltpu.VMEM((CHUNK,), jnp.int32),
            pltpu.VMEM((CHUNK,), jnp.int32),
            pltpu.VMEM((CHUNK, 128), jnp.float32),
            pltpu.SemaphoreType.DMA,
        ],
    )
    return f(nf_p, src, dst)


def _drain_pad(pend, tl):
    dummy = lax.broadcast(jnp.int32(PKDUM), (16,))
    for j in range(FIRE // 16):
        pend[pl.ds(tl + j * 16, 16)] = dummy


def _consume(cpk_hbm, cnt_hbm, cpkv, cntv, pend, wid, fire):
    def chunk(i, tl):
        ci = lax.rem(i + wid * 15, NSUP)
        pltpu.sync_copy(cpk_hbm.at[pl.ds(wid * (GPT * 16) + ci * SUPW, SUPW)],
                        cpkv)
        pltpu.sync_copy(cnt_hbm.at[pl.ds(wid * GPT + ci * SUPG, SUPG)],
                        cntv.at[pl.ds(0, SUPG)])
        cvs = [cntv[pl.ds(k * 16, 16)] for k in range(SUPG // 16 + 1)]

        def maybe_fire(j, tl):
            return lax.cond(tl >= FIRE, fire, lambda t: t, tl)

        for j in range(SUPG):
            pend[pl.ds(tl, 16)] = cpkv[pl.ds(j * 16, 16)]
            tl = tl + cvs[j // 16][j % 16]
            if j % 8 == 7:
                tl = lax.fori_loop(0, 4, maybe_fire, tl)
        return tl

    tl = lax.fori_loop(0, NSUP, chunk, 0)
    _drain_pad(pend, tl)
    lax.cond(tl > 0, fire, lambda t: t, tl)


def _seg_rows_call(table, cpk, cnt, zacc):
    def body(table_hbm, cpk_hbm, cnt_hbm, zacc_hbm, out_hbm,
             cpkv, cntv, pend, gbuf, rows, acc, sem):
        c = lax.axis_index("c")
        s = lax.axis_index("s")
        wid = s * 2 + c
        wbase = wid * WIN
        pltpu.sync_copy(zacc_hbm, acc)

        def fire(tl):
            for g in range(FIRE // 16):
                pk = pend[pl.ds(g * 16, 16)]
                gbuf[pl.ds(g * 16, 16)] = lax.shift_right_logical(pk, 9)
            pltpu.async_copy(table_hbm.at[gbuf], rows, sem).wait()
            for g in range(FIRE // 16):
                lvec = pend[pl.ds(g * 16, 16)] & 511
                for lane in range(16):
                    lj = lvec[lane]
                    for k in range(HID // 16):
                        plsc.addupdate(acc.at[lj, pl.ds(k * 16, 16)],
                                       rows[g * 16 + lane, pl.ds(k * 16, 16)])
            for g in range(PSHIFT // 16):
                pend[pl.ds(g * 16, 16)] = pend[pl.ds(g * 16 + FIRE, 16)]
            return tl - FIRE

        _consume(cpk_hbm, cnt_hbm, cpkv, cntv, pend, wid, fire)
        pltpu.sync_copy(acc.at[pl.ds(0, WIN)], out_hbm.at[pl.ds(wbase, WIN)])

    f = pl.kernel(
        body,
        mesh=_mesh(),
        out_type=jax.ShapeDtypeStruct((NP, HID), jnp.float32),
        scratch_types=[
            pltpu.VMEM((SUPW,), jnp.int32),
            pltpu.VMEM((SUPG + 8,), jnp.int32),
            pltpu.VMEM((PCAP,), jnp.int32),
            pltpu.VMEM((FIRE,), jnp.int32),
            pltpu.VMEM((FIRE, HID), jnp.float32),
            pltpu.VMEM((ACC_R, HID), jnp.float32),
            pltpu.SemaphoreType.DMA,
        ],
    )
    return f(table, cpk, cnt, zacc)


def _seg_count_call(cpk, cnt, zdeg):
    def body(cpk_hbm, cnt_hbm, zdeg_hbm, deg_hbm, cpkv, cntv, pend, dacc, sem):
        c = lax.axis_index("c")
        s = lax.axis_index("s")
        wid = s * 2 + c
        wbase = wid * WIN
        pltpu.sync_copy(zdeg_hbm, dacc)
        iota16 = lax.iota(jnp.int32, 16)
        onehot = jnp.where(iota16 == 0, 1.0, 0.0)

        def fire(tl):
            for g in range(FIRE // 16):
                lvec = pend[pl.ds(g * 16, 16)] & 511
                for lane in range(16):
                    plsc.addupdate(dacc.at[lvec[lane]], onehot)
            for g in range(PSHIFT // 16):
                pend[pl.ds(g * 16, 16)] = pend[pl.ds(g * 16 + FIRE, 16)]
            return tl - FIRE

        _consume(cpk_hbm, cnt_hbm, cpkv, cntv, pend, wid, fire)
        pltpu.sync_copy(dacc.at[pl.ds(0, WIN)], deg_hbm.at[pl.ds(wbase, WIN)])

    f = pl.kernel(
        body,
        mesh=_mesh(),
        out_type=jax.ShapeDtypeStruct((NP, 16), jnp.float32),
        scratch_types=[
            pltpu.VMEM((SUPW,), jnp.int32),
            pltpu.VMEM((SUPG + 8,), jnp.int32),
            pltpu.VMEM((PCAP,), jnp.int32),
            pltpu.VMEM((ACC_R, 16), jnp.float32),
            pltpu.SemaphoreType.DMA,
        ],
    )
    return f(cpk, cnt, zdeg)


def _compact_call(key2, vals2):
    RB = 2000

    def body(k_r, v_r, out_r):
        t = pl.program_id(0)
        l = k_r[...] - t * WIN
        ok = (l >= 0) & (l < WIN)
        okf = ok.astype(jnp.float32)
        iota_l = lax.broadcasted_iota(jnp.int32, (RB, 16), 1)
        mstrict = (lax.broadcasted_iota(jnp.int32, (16, 16), 0)
                   < lax.broadcasted_iota(jnp.int32, (16, 16), 1))
        rank = jnp.dot(okf, mstrict.astype(jnp.float32),
                       preferred_element_type=jnp.float32)
        vf = v_r[...].astype(jnp.float32)
        lf = l.astype(jnp.float32)
        vc = jnp.zeros((RB, 16), jnp.float32)
        lc = jnp.zeros((RB, 16), jnp.float32)
        iota_f = iota_l.astype(jnp.float32)
        for i in range(16):
            sel = okf[:, i:i + 1] * (rank[:, i:i + 1] == iota_f).astype(jnp.float32)
            vc = vc + sel * vf[:, i:i + 1]
            lc = lc + sel * lf[:, i:i + 1]
        cntr = jnp.sum(okf, axis=1, keepdims=True)
        valid = iota_l < cntr.astype(jnp.int32)
        packed = jnp.where(valid,
                           vc.astype(jnp.int32) * 512 + lc.astype(jnp.int32),
                           PKDUM)
        out_r[...] = packed[None]

    return pl.pallas_call(
        body,
        grid=(NW, GPT // RB),
        in_specs=[
            pl.BlockSpec((RB, 16), lambda t, b: (b, 0)),
            pl.BlockSpec((RB, 16), lambda t, b: (b, 0)),
        ],
        out_specs=pl.BlockSpec((1, RB, 16), lambda t, b: (t, b, 0)),
        out_shape=jax.ShapeDtypeStruct((NW, GPT, 16), jnp.int32),
    )(key2, vals2)


def _group_counts_call(key2T):
    def body(kt_r, out_r):
        t = pl.program_id(0)
        l = kt_r[...] - t * WIN
        okf = ((l >= 0) & (l < WIN)).astype(jnp.float32)
        out_r[...] = jnp.sum(okf, axis=0, keepdims=True)[None].astype(jnp.int32)

    return pl.pallas_call(
        body,
        grid=(NW,),
        in_specs=[pl.BlockSpec((16, GPT), lambda t: (0, 0))],
        out_specs=pl.BlockSpec((1, 1, GPT), lambda t: (t, 0, 0)),
        out_shape=jax.ShapeDtypeStruct((NW, 1, GPT), jnp.int32),
    )(key2T)


def _silu(x):
    return x * (1.0 / (1.0 + jnp.exp(-x)))


def _edge_mlp_call(hs, hd, ef, Ws, Wd, We, wr, be0, We1, be1):
    def body(hs_r, hd_r, ef_r, Ws_r, Wd_r, We_r, wr_r, be0_r,
             We1_r, be1_r, out_r):
        xd = hs_r[:, 96:112] - hd_r[:, 96:112]
        rad = jnp.sum(xd * xd, axis=1, keepdims=True)
        z = (jnp.dot(hs_r[...], Ws_r[...], preferred_element_type=jnp.float32)
             + jnp.dot(hd_r[...], Wd_r[...], preferred_element_type=jnp.float32)
             + jnp.dot(ef_r[...], We_r[...], preferred_element_type=jnp.float32)
             + rad * wr_r[...] + be0_r[...])
        t = _silu(z)
        z2 = jnp.dot(t, We1_r[...], preferred_element_type=jnp.float32) + be1_r[...]
        out_r[...] = _silu(z2)

    full = lambda shape: pl.BlockSpec(shape, lambda i: (0, 0))
    return pl.pallas_call(
        body,
        grid=(E // BE,),
        in_specs=[
            pl.BlockSpec((BE, 128), lambda i: (i, 0)),
            pl.BlockSpec((BE, 128), lambda i: (i, 0)),
            pl.BlockSpec((BE, 8), lambda i: (i, 0)),
            full((128, HID)),
            full((128, HID)),
            full((8, HID)),
            full((1, HID)),
            full((1, HID)),
            full((HID, HID)),
            full((1, HID)),
        ],
        out_specs=pl.BlockSpec((BE, HID), lambda i: (i, 0)),
        out_shape=jax.ShapeDtypeStruct((E, HID), jnp.float32),
    )(hs, hd, ef, Ws, Wd, We, wr, be0, We1, be1)


def _node_mlp_call(nf_p, hn, deg_in, deg_out, Wa, Wb, bn0, Wn1, bn1):
    def body(nf_r, hn_r, din_r, dout_r, Wa_r, Wb_r, bn0_r, Wn1_r, bn1_r,
             h0_r, feat_r, nin_r, nout_r):
        z = (jnp.dot(nf_r[...], Wa_r[...], preferred_element_type=jnp.float32)
             + jnp.dot(hn_r[...], Wb_r[...], preferred_element_type=jnp.float32)
             + bn0_r[...])
        t = _silu(z)
        h = jnp.dot(t, Wn1_r[...], preferred_element_type=jnp.float32) + bn1_r[...]
        h0_r[...] = h
        nin_r[...] = lax.rsqrt(jnp.maximum(din_r[:, 0:1], 1.0))
        no = lax.rsqrt(jnp.maximum(dout_r[:, 0:1], 1.0))
        nout_r[...] = no
        feat_r[...] = h * no

    full = lambda shape: pl.BlockSpec(shape, lambda i: (0, 0))
    return pl.pallas_call(
        body,
        grid=(NP // BN,),
        in_specs=[
            pl.BlockSpec((BN, 128), lambda i: (i, 0)),
            pl.BlockSpec((BN, HID), lambda i: (i, 0)),
            pl.BlockSpec((BN, 16), lambda i: (i, 0)),
            pl.BlockSpec((BN, 16), lambda i: (i, 0)),
            full((128, HID)),
            full((HID, HID)),
            full((1, HID)),
            full((HID, HID)),
            full((1, HID)),
        ],
        out_specs=[
            pl.BlockSpec((BN, HID), lambda i: (i, 0)),
            pl.BlockSpec((BN, HID), lambda i: (i, 0)),
            pl.BlockSpec((BN, 1), lambda i: (i, 0)),
            pl.BlockSpec((BN, 1), lambda i: (i, 0)),
        ],
        out_shape=[
            jax.ShapeDtypeStruct((NP, HID), jnp.float32),
            jax.ShapeDtypeStruct((NP, HID), jnp.float32),
            jax.ShapeDtypeStruct((NP, 1), jnp.float32),
            jax.ShapeDtypeStruct((NP, 1), jnp.float32),
        ],
    )(nf_p, hn, deg_in, deg_out, Wa, Wb, bn0, Wn1, bn1)


def _gcn_layer_call(agg, res, nin, nout, Wg, bg, beta):
    def body(agg_r, res_r, nin_r, nout_r, Wg_r, bg_r, h_r, feat_r):
        rst = agg_r[...] * (nin_r[...] * (1.0 - ALPHA)) + ALPHA * res_r[...]
        y = ((1.0 - beta) * rst
             + beta * jnp.dot(rst, Wg_r[...], preferred_element_type=jnp.float32)
             + bg_r[...])
        h = _silu(y)
        h_r[...] = h
        feat_r[...] = h * nout_r[...]

    full = lambda shape: pl.BlockSpec(shape, lambda i: (0, 0))
    return pl.pallas_call(
        body,
        grid=(NP // BN,),
        in_specs=[
            pl.BlockSpec((BN, HID), lambda i: (i, 0)),
            pl.BlockSpec((BN, HID), lambda i: (i, 0)),
            pl.BlockSpec((BN, 1), lambda i: (i, 0)),
            pl.BlockSpec((BN, 1), lambda i: (i, 0)),
            full((HID, HID)),
            full((1, HID)),
        ],
        out_specs=[
            pl.BlockSpec((BN, HID), lambda i: (i, 0)),
            pl.BlockSpec((BN, HID), lambda i: (i, 0)),
        ],
        out_shape=[
            jax.ShapeDtypeStruct((NP, HID), jnp.float32),
            jax.ShapeDtypeStruct((NP, HID), jnp.float32),
        ],
    )(agg, res, nin, nout, Wg, bg)


def kernel(node_feat, coord, edge_feat, params, edge_index):
    f32 = jnp.float32
    src = edge_index[0]
    dst = edge_index[1]

    nf_p = (jnp.zeros((NP, 128), f32)
            .at[:N, :82].set(node_feat)
            .at[:N, 96:99].set(coord))
    ef_p = jnp.zeros((E, 8), f32).at[:, :6].set(edge_feat)

    We0 = params["We0"]
    Ws = jnp.zeros((128, HID), f32).at[:82].set(We0[:82])
    Wd = jnp.zeros((128, HID), f32).at[:82].set(We0[82:164])
    wr = We0[164:165]
    We = jnp.zeros((8, HID), f32).at[:6].set(We0[165:171])
    be0 = params["be0"][None, :]
    We1 = params["We1"]
    be1 = params["be1"][None, :]
    Wn0 = params["Wn0"]
    Wa = jnp.zeros((128, HID), f32).at[:82].set(Wn0[:82])
    Wb = Wn0[82:338]
    bn0 = params["bn0"][None, :]
    Wn1 = params["Wn1"]
    bn1 = params["bn1"][None, :]

    zacc = jnp.zeros((ACC_R, HID), f32)
    zdeg = jnp.zeros((ACC_R, 16), f32)
    iota_e = jnp.arange(E, dtype=jnp.int32)

    dst2 = dst.reshape(GPT, 16)
    src2 = src.reshape(GPT, 16)
    cpk_msg = _compact_call(dst2, iota_e.reshape(GPT, 16)).reshape(NW * GPT * 16)
    cpk_gcn = _compact_call(dst2, src2).reshape(NW * GPT * 16)
    cpk_src = _compact_call(src2, src2).reshape(NW * GPT * 16)
    cnt_dst = _group_counts_call(jnp.transpose(dst2)).reshape(NW * GPT)
    cnt_src = _group_counts_call(jnp.transpose(src2)).reshape(NW * GPT)

    hs, hd = _gather_edges_call(nf_p, src, dst)
    msg = _edge_mlp_call(hs, hd, ef_p, Ws, Wd, We, wr, be0, We1, be1)
    hn = _seg_rows_call(msg, cpk_msg, cnt_dst, zacc)
    deg_in = _seg_count_call(cpk_msg, cnt_dst, zdeg)
    deg_out = _seg_count_call(cpk_src, cnt_src, zdeg)
    h0, feat, nin, nout = _node_mlp_call(nf_p, hn, deg_in, deg_out,
                                         Wa, Wb, bn0, Wn1, bn1)

    h = h0
    for i in range(8):
        beta = math.log(1.0 / (i + 1) + 1.0)
        agg = _seg_rows_call(feat, cpk_gcn, cnt_dst, zacc)
        h, feat = _gcn_layer_call(agg, h0, nin, nout, params["Wg"][i],
                                  params["bg"][i][None, :], beta)
    return h[:N]


# speedup vs baseline: 1.5273x; 1.5273x over previous
import math

import jax
import jax.numpy as jnp
from jax import lax
from jax.experimental import pallas as pl
from jax.experimental.pallas import tpu as pltpu
from jax.experimental.pallas import tpu_sc as plsc

N = 10000
E = 320000
NP = 10240
NW = 32
WIN = NP // NW
ACC_R = WIN + 8
PKDUM = WIN
EPW = E // NW
CHUNK = 80
NCHUNK = EPW // CHUNK
GPT = E // 16
SUPG = 40
SUPW = SUPG * 16
NSUP = GPT // SUPG
FIRE = 32
PCAP = 176
PSHIFT = PCAP - FIRE
HID = 256
ALPHA = 0.5
BE = 512
BN = 512


def _mesh():
    return plsc.VectorSubcoreMesh(core_axis_name="c", subcore_axis_name="s")


def _gather_edges_call(nf_p, src, dst):
    def body(nf_hbm, src_hbm, dst_hbm, hs_hbm, hd_hbm, srcv, dstv, nbuf, sem):
        wid = lax.axis_index("s") * 2 + lax.axis_index("c")

        def step(i, carry):
            base = wid * EPW + i * CHUNK
            pltpu.sync_copy(src_hbm.at[pl.ds(base, CHUNK)], srcv)
            pltpu.sync_copy(dst_hbm.at[pl.ds(base, CHUNK)], dstv)
            pltpu.async_copy(nf_hbm.at[srcv], nbuf, sem).wait()
            pltpu.sync_copy(nbuf, hs_hbm.at[pl.ds(base, CHUNK)])
            pltpu.async_copy(nf_hbm.at[dstv], nbuf, sem).wait()
            pltpu.sync_copy(nbuf, hd_hbm.at[pl.ds(base, CHUNK)])
            return carry

        lax.fori_loop(0, NCHUNK, step, 0)

    f = pl.kernel(
        body,
        mesh=_mesh(),
        out_type=(
            jax.ShapeDtypeStruct((E, 128), jnp.float32),
            jax.ShapeDtypeStruct((E, 128), jnp.float32),
        ),
        scratch_types=[
            pltpu.VMEM((CHUNK,), jnp.int32),
            pltpu.VMEM((CHUNK,), jnp.int32),
            pltpu.VMEM((CHUNK, 128), jnp.float32),
            pltpu.SemaphoreType.DMA,
        ],
    )
    return f(nf_p, src, dst)


def _drain_pad(pend, tl):
    dummy = lax.broadcast(jnp.int32(PKDUM), (16,))
    for j in range(FIRE // 16):
        pend[pl.ds(tl + j * 16, 16)] = dummy


def _consume(cpk_hbm, cnt_hbm, cpkv, cntv, pend, wid, fire):
    def chunk(i, tl):
        ci = lax.rem(i + wid * 15, NSUP)
        pltpu.sync_copy(cpk_hbm.at[pl.ds(wid * (GPT * 16) + ci * SUPW, SUPW)],
                        cpkv)
        pltpu.sync_copy(cnt_hbm.at[pl.ds(wid * GPT + ci * SUPG, SUPG)],
                        cntv.at[pl.ds(0, SUPG)])
        cvs = [cntv[pl.ds(k * 16, 16)] for k in range(SUPG // 16 + 1)]

        def maybe_fire(j, tl):
            return lax.cond(tl >= FIRE, fire, lambda t: t, tl)

        for j in range(SUPG):
            pend[pl.ds(tl, 16)] = cpkv[pl.ds(j * 16, 16)]
            tl = tl + cvs[j // 16][j % 16]
            if j % 8 == 7:
                tl = lax.fori_loop(0, 4, maybe_fire, tl)
        return tl

    tl = lax.fori_loop(0, NSUP, chunk, 0)
    _drain_pad(pend, tl)
    lax.cond(tl > 0, fire, lambda t: t, tl)


def _seg_rows_call(table, cpk, cnt, zacc):
    def body(table_hbm, cpk_hbm, cnt_hbm, zacc_hbm, out_hbm,
             cpkv, cntv, pend, gbuf, rows, acc, sem):
        c = lax.axis_index("c")
        s = lax.axis_index("s")
        wid = s * 2 + c
        wbase = wid * WIN
        pltpu.sync_copy(zacc_hbm, acc)

        def fire(tl):
            for g in range(FIRE // 16):
                pk = pend[pl.ds(g * 16, 16)]
                gbuf[pl.ds(g * 16, 16)] = lax.shift_right_logical(pk, 9)
            pltpu.async_copy(table_hbm.at[gbuf], rows, sem).wait()
            for g in range(FIRE // 16):
                lvec = pend[pl.ds(g * 16, 16)] & 511
                for lane in range(16):
                    lj = lvec[lane]
                    for k in range(HID // 16):
                        plsc.addupdate(acc.at[lj, pl.ds(k * 16, 16)],
                                       rows[g * 16 + lane, pl.ds(k * 16, 16)])
            for g in range(PSHIFT // 16):
                pend[pl.ds(g * 16, 16)] = pend[pl.ds(g * 16 + FIRE, 16)]
            return tl - FIRE

        _consume(cpk_hbm, cnt_hbm, cpkv, cntv, pend, wid, fire)
        pltpu.sync_copy(acc.at[pl.ds(0, WIN)], out_hbm.at[pl.ds(wbase, WIN)])

    f = pl.kernel(
        body,
        mesh=_mesh(),
        out_type=jax.ShapeDtypeStruct((NP, HID), jnp.float32),
        scratch_types=[
            pltpu.VMEM((SUPW,), jnp.int32),
            pltpu.VMEM((SUPG + 8,), jnp.int32),
            pltpu.VMEM((PCAP,), jnp.int32),
            pltpu.VMEM((FIRE,), jnp.int32),
            pltpu.VMEM((FIRE, HID), jnp.float32),
            pltpu.VMEM((ACC_R, HID), jnp.float32),
            pltpu.SemaphoreType.DMA,
        ],
    )
    return f(table, cpk, cnt, zacc)


def _seg_count_call(cpk, cnt, zdeg):
    def body(cpk_hbm, cnt_hbm, zdeg_hbm, deg_hbm, cpkv, cntv, pend, dacc, sem):
        c = lax.axis_index("c")
        s = lax.axis_index("s")
        wid = s * 2 + c
        wbase = wid * WIN
        pltpu.sync_copy(zdeg_hbm, dacc)
        iota16 = lax.iota(jnp.int32, 16)
        onehot = jnp.where(iota16 == 0, 1.0, 0.0)

        def fire(tl):
            for g in range(FIRE // 16):
                lvec = pend[pl.ds(g * 16, 16)] & 511
                for lane in range(16):
                    plsc.addupdate(dacc.at[lvec[lane]], onehot)
            for g in range(PSHIFT // 16):
                pend[pl.ds(g * 16, 16)] = pend[pl.ds(g * 16 + FIRE, 16)]
            return tl - FIRE

        _consume(cpk_hbm, cnt_hbm, cpkv, cntv, pend, wid, fire)
        pltpu.sync_copy(dacc.at[pl.ds(0, WIN)], deg_hbm.at[pl.ds(wbase, WIN)])

    f = pl.kernel(
        body,
        mesh=_mesh(),
        out_type=jax.ShapeDtypeStruct((NP, 16), jnp.float32),
        scratch_types=[
            pltpu.VMEM((SUPW,), jnp.int32),
            pltpu.VMEM((SUPG + 8,), jnp.int32),
            pltpu.VMEM((PCAP,), jnp.int32),
            pltpu.VMEM((ACC_R, 16), jnp.float32),
            pltpu.SemaphoreType.DMA,
        ],
    )
    return f(cpk, cnt, zdeg)


def _compact_call(key2T, vals2T):
    def body(kt_r, vt_r, out_r):
        t = pl.program_id(0)
        l = kt_r[...] - t * WIN
        ok = (l >= 0) & (l < WIN)
        okf = ok.astype(jnp.float32)
        mstrict = (lax.broadcasted_iota(jnp.int32, (16, 16), 1)
                   < lax.broadcasted_iota(jnp.int32, (16, 16), 0))
        rank = jnp.dot(mstrict.astype(jnp.float32), okf,
                       preferred_element_type=jnp.float32)
        vf = vt_r[...].astype(jnp.float32)
        lf = l.astype(jnp.float32)
        cnt = jnp.sum(okf, axis=0, keepdims=True)
        rows = []
        for p in range(16):
            selp = okf * (rank == p).astype(jnp.float32)
            vp = jnp.sum(selp * vf, axis=0, keepdims=True)
            lp = jnp.sum(selp * lf, axis=0, keepdims=True)
            rowp = jnp.where(p < cnt,
                             vp.astype(jnp.int32) * 512 + lp.astype(jnp.int32),
                             PKDUM)
            rows.append(rowp)
        out_r[...] = jnp.concatenate(rows, axis=0)[None]

    return pl.pallas_call(
        body,
        grid=(NW,),
        in_specs=[
            pl.BlockSpec((16, GPT), lambda t: (0, 0)),
            pl.BlockSpec((16, GPT), lambda t: (0, 0)),
        ],
        out_specs=pl.BlockSpec((1, 16, GPT), lambda t: (t, 0, 0)),
        out_shape=jax.ShapeDtypeStruct((NW, 16, GPT), jnp.int32),
    )(key2T, vals2T)


def _group_counts_call(key2T):
    def body(kt_r, out_r):
        t = pl.program_id(0)
        l = kt_r[...] - t * WIN
        okf = ((l >= 0) & (l < WIN)).astype(jnp.float32)
        out_r[...] = jnp.sum(okf, axis=0, keepdims=True)[None].astype(jnp.int32)

    return pl.pallas_call(
        body,
        grid=(NW,),
        in_specs=[pl.BlockSpec((16, GPT), lambda t: (0, 0))],
        out_specs=pl.BlockSpec((1, 1, GPT), lambda t: (t, 0, 0)),
        out_shape=jax.ShapeDtypeStruct((NW, 1, GPT), jnp.int32),
    )(key2T)


def _silu(x):
    return x * (1.0 / (1.0 + jnp.exp(-x)))


def _edge_mlp_call(hs, hd, ef, Ws, Wd, We, wr, be0, We1, be1):
    def body(hs_r, hd_r, ef_r, Ws_r, Wd_r, We_r, wr_r, be0_r,
             We1_r, be1_r, out_r):
        xd = hs_r[:, 96:112] - hd_r[:, 96:112]
        rad = jnp.sum(xd * xd, axis=1, keepdims=True)
        z = (jnp.dot(hs_r[...], Ws_r[...], preferred_element_type=jnp.float32)
             + jnp.dot(hd_r[...], Wd_r[...], preferred_element_type=jnp.float32)
             + jnp.dot(ef_r[...], We_r[...], preferred_element_type=jnp.float32)
             + rad * wr_r[...] + be0_r[...])
        t = _silu(z)
        z2 = jnp.dot(t, We1_r[...], preferred_element_type=jnp.float32) + be1_r[...]
        out_r[...] = _silu(z2)

    full = lambda shape: pl.BlockSpec(shape, lambda i: (0, 0))
    return pl.pallas_call(
        body,
        grid=(E // BE,),
        in_specs=[
            pl.BlockSpec((BE, 128), lambda i: (i, 0)),
            pl.BlockSpec((BE, 128), lambda i: (i, 0)),
            pl.BlockSpec((BE, 8), lambda i: (i, 0)),
            full((128, HID)),
            full((128, HID)),
            full((8, HID)),
            full((1, HID)),
            full((1, HID)),
            full((HID, HID)),
            full((1, HID)),
        ],
        out_specs=pl.BlockSpec((BE, HID), lambda i: (i, 0)),
        out_shape=jax.ShapeDtypeStruct((E, HID), jnp.float32),
    )(hs, hd, ef, Ws, Wd, We, wr, be0, We1, be1)


def _node_mlp_call(nf_p, hn, deg_in, deg_out, Wa, Wb, bn0, Wn1, bn1):
    def body(nf_r, hn_r, din_r, dout_r, Wa_r, Wb_r, bn0_r, Wn1_r, bn1_r,
             h0_r, feat_r, nin_r, nout_r):
        z = (jnp.dot(nf_r[...], Wa_r[...], preferred_element_type=jnp.float32)
             + jnp.dot(hn_r[...], Wb_r[...], preferred_element_type=jnp.float32)
             + bn0_r[...])
        t = _silu(z)
        h = jnp.dot(t, Wn1_r[...], preferred_element_type=jnp.float32) + bn1_r[...]
        h0_r[...] = h
        nin_r[...] = lax.rsqrt(jnp.maximum(din_r[:, 0:1], 1.0))
        no = lax.rsqrt(jnp.maximum(dout_r[:, 0:1], 1.0))
        nout_r[...] = no
        feat_r[...] = h * no

    full = lambda shape: pl.BlockSpec(shape, lambda i: (0, 0))
    return pl.pallas_call(
        body,
        grid=(NP // BN,),
        in_specs=[
            pl.BlockSpec((BN, 128), lambda i: (i, 0)),
            pl.BlockSpec((BN, HID), lambda i: (i, 0)),
            pl.BlockSpec((BN, 16), lambda i: (i, 0)),
            pl.BlockSpec((BN, 16), lambda i: (i, 0)),
            full((128, HID)),
            full((HID, HID)),
            full((1, HID)),
            full((HID, HID)),
            full((1, HID)),
        ],
        out_specs=[
            pl.BlockSpec((BN, HID), lambda i: (i, 0)),
            pl.BlockSpec((BN, HID), lambda i: (i, 0)),
            pl.BlockSpec((BN, 1), lambda i: (i, 0)),
            pl.BlockSpec((BN, 1), lambda i: (i, 0)),
        ],
        out_shape=[
            jax.ShapeDtypeStruct((NP, HID), jnp.float32),
            jax.ShapeDtypeStruct((NP, HID), jnp.float32),
            jax.ShapeDtypeStruct((NP, 1), jnp.float32),
            jax.ShapeDtypeStruct((NP, 1), jnp.float32),
        ],
    )(nf_p, hn, deg_in, deg_out, Wa, Wb, bn0, Wn1, bn1)


def _gcn_layer_call(agg, res, nin, nout, Wg, bg, beta):
    def body(agg_r, res_r, nin_r, nout_r, Wg_r, bg_r, h_r, feat_r):
        rst = agg_r[...] * (nin_r[...] * (1.0 - ALPHA)) + ALPHA * res_r[...]
        y = ((1.0 - beta) * rst
             + beta * jnp.dot(rst, Wg_r[...], preferred_element_type=jnp.float32)
             + bg_r[...])
        h = _silu(y)
        h_r[...] = h
        feat_r[...] = h * nout_r[...]

    full = lambda shape: pl.BlockSpec(shape, lambda i: (0, 0))
    return pl.pallas_call(
        body,
        grid=(NP // BN,),
        in_specs=[
            pl.BlockSpec((BN, HID), lambda i: (i, 0)),
            pl.BlockSpec((BN, HID), lambda i: (i, 0)),
            pl.BlockSpec((BN, 1), lambda i: (i, 0)),
            pl.BlockSpec((BN, 1), lambda i: (i, 0)),
            full((HID, HID)),
            full((1, HID)),
        ],
        out_specs=[
            pl.BlockSpec((BN, HID), lambda i: (i, 0)),
            pl.BlockSpec((BN, HID), lambda i: (i, 0)),
        ],
        out_shape=[
            jax.ShapeDtypeStruct((NP, HID), jnp.float32),
            jax.ShapeDtypeStruct((NP, HID), jnp.float32),
        ],
    )(agg, res, nin, nout, Wg, bg)


def kernel(node_feat, coord, edge_feat, params, edge_index):
    f32 = jnp.float32
    src = edge_index[0]
    dst = edge_index[1]

    nf_p = (jnp.zeros((NP, 128), f32)
            .at[:N, :82].set(node_feat)
            .at[:N, 96:99].set(coord))
    ef_p = jnp.zeros((E, 8), f32).at[:, :6].set(edge_feat)

    We0 = params["We0"]
    Ws = jnp.zeros((128, HID), f32).at[:82].set(We0[:82])
    Wd = jnp.zeros((128, HID), f32).at[:82].set(We0[82:164])
    wr = We0[164:165]
    We = jnp.zeros((8, HID), f32).at[:6].set(We0[165:171])
    be0 = params["be0"][None, :]
    We1 = params["We1"]
    be1 = params["be1"][None, :]
    Wn0 = params["Wn0"]
    Wa = jnp.zeros((128, HID), f32).at[:82].set(Wn0[:82])
    Wb = Wn0[82:338]
    bn0 = params["bn0"][None, :]
    Wn1 = params["Wn1"]
    bn1 = params["bn1"][None, :]

    zacc = jnp.zeros((ACC_R, HID), f32)
    zdeg = jnp.zeros((ACC_R, 16), f32)
    iota_e = jnp.arange(E, dtype=jnp.int32)

    dst2T = jnp.transpose(dst.reshape(GPT, 16))
    src2T = jnp.transpose(src.reshape(GPT, 16))
    iota2T = jnp.transpose(iota_e.reshape(GPT, 16))

    def _stream(cT):
        return jnp.swapaxes(cT, 1, 2).reshape(NW * GPT * 16)

    cpk_msg = _stream(_compact_call(dst2T, iota2T))
    cpk_gcn = _stream(_compact_call(dst2T, src2T))
    cpk_src = _stream(_compact_call(src2T, src2T))
    cnt_dst = _group_counts_call(dst2T).reshape(NW * GPT)
    cnt_src = _group_counts_call(src2T).reshape(NW * GPT)

    hs, hd = _gather_edges_call(nf_p, src, dst)
    msg = _edge_mlp_call(hs, hd, ef_p, Ws, Wd, We, wr, be0, We1, be1)
    hn = _seg_rows_call(msg, cpk_msg, cnt_dst, zacc)
    deg_in = _seg_count_call(cpk_msg, cnt_dst, zdeg)
    deg_out = _seg_count_call(cpk_src, cnt_src, zdeg)
    h0, feat, nin, nout = _node_mlp_call(nf_p, hn, deg_in, deg_out,
                                         Wa, Wb, bn0, Wn1, bn1)

    h = h0
    for i in range(8):
        beta = math.log(1.0 / (i + 1) + 1.0)
        agg = _seg_rows_call(feat, cpk_gcn, cnt_dst, zacc)
        h, feat = _gcn_layer_call(agg, h0, nin, nout, params["Wg"][i],
                                  params["bg"][i][None, :], beta)
    return h[:N]
